# Initial kernel scaffold; baseline (speedup 1.0000x reference)
import jax
import jax.numpy as jnp
from jax.experimental import pallas as pl


def kernel(node_emb, rel_emb, edge_index, W_loop, W_in, W_out, W_rel, loop_rel, bias):
    raise NotImplementedError("write your pallas kernel here")


# speedup vs baseline: 2.5033x; 2.5033x over previous
import functools

import jax
import jax.numpy as jnp
from jax import lax
from jax.experimental import pallas as pl
from jax.experimental.pallas import tpu as pltpu
from jax.experimental.pallas import tpu_sc as plsc

N = 10000
E = 160000
D = 256
H = 128
NPAD = 10240
EPAD = 163840
CH = 128
ROWS = EPAD // 32 // CH
ACCR = NPAD
NB = 1024
NBO = 1000
TRASH = N

_MESH = plsc.VectorSubcoreMesh(core_axis_name="c", subcore_axis_name="s",
                               num_cores=2, num_subcores=16)


def _deg_body(s2, t2, zeros_h, deg_out, slab, ones_v, hist):
    cid = lax.axis_index("c")
    sid = lax.axis_index("s")

    def fill_ones(i, _):
        for g in range(H // 16):
            ones_v[i, pl.ds(g * 16, 16)] = jnp.full((16,), 1.0, jnp.float32)
        return 0

    lax.fori_loop(0, CH, fill_ones, 0)
    pltpu.sync_copy(zeros_h, hist.at[pl.ds(sid * (NPAD // 16), NPAD // 16)])
    plsc.subcore_barrier()

    def do_hist(src):
        pltpu.sync_copy(src.at[pl.ds(sid * (2 * ROWS), 2 * ROWS)], slab)

        def body(j, _):
            pltpu.sync_copy(ones_v, hist.at[slab.at[j]], add=True)
            return 0

        lax.fori_loop(0, 2 * ROWS, body, 0)

    pl.when(cid == 0)(lambda: do_hist(s2))
    pl.when(cid == 1)(lambda: do_hist(t2))
    plsc.subcore_barrier()
    sl = pl.ds(sid * (NPAD // 16), NPAD // 16)
    pltpu.sync_copy(hist.at[sl], deg_out.at[cid, sl])


_deg_kernel = functools.partial(
    pl.kernel,
    out_type=jax.ShapeDtypeStruct((2, NPAD, H), jnp.float32),
    mesh=_MESH,
    scratch_types=[
        pltpu.VMEM((2 * ROWS, CH), jnp.int32),
        pltpu.VMEM((CH, H), jnp.float32),
        pltpu.VMEM_SHARED((NPAD, H), jnp.float32),
    ],
)(_deg_body)


def _edge_body(ns0, ns1, nt0, nt1, rel0, rel1, s2, r2, t2, zeros_h, pre_out,
               s_slab, r_slab, t_slab, nbuf, rbuf, sem, acc):
    cid = lax.axis_index("c")
    sid = lax.axis_index("s")
    w = sid * 2 + cid

    pltpu.sync_copy(s2.at[pl.ds(w * ROWS, ROWS)], s_slab)
    pltpu.sync_copy(r2.at[pl.ds(w * ROWS, ROWS)], r_slab)
    pltpu.sync_copy(t2.at[pl.ds(w * ROWS, ROWS)], t_slab)

    for d, h, tab, rtab, g_slab, d_slab in (
            (0, 0, ns0, rel0, s_slab, t_slab),
            (0, 1, ns1, rel1, s_slab, t_slab),
            (1, 0, nt0, rel0, t_slab, s_slab),
            (1, 1, nt1, rel1, t_slab, s_slab)):
        pltpu.sync_copy(zeros_h, acc.at[pl.ds(sid * (ACCR // 16), ACCR // 16)])
        plsc.subcore_barrier()

        def chunk(j, _):
            cp1 = pltpu.async_copy(tab.at[g_slab.at[j]], nbuf, sem)
            cp2 = pltpu.async_copy(rtab.at[r_slab.at[j]], rbuf, sem)
            cp1.wait()
            cp2.wait()

            def mul_body(i, _2):
                for g in range(H // 16):
                    sl = pl.ds(g * 16, 16)
                    nbuf[i, sl] = nbuf[i, sl] * rbuf[i, sl]
                return 0

            lax.fori_loop(0, CH, mul_body, 0)
            pltpu.sync_copy(nbuf, acc.at[d_slab.at[j]], add=True)
            return 0

        lax.fori_loop(0, ROWS, chunk, 0)
        plsc.subcore_barrier()
        sl = pl.ds(sid * (ACCR // 16), ACCR // 16)
        pltpu.sync_copy(acc.at[sl], pre_out.at[d, cid, h, sl])
        plsc.subcore_barrier()


_edge_kernel = functools.partial(
    pl.kernel,
    out_type=jax.ShapeDtypeStruct((2, 2, 2, ACCR, H), jnp.float32),
    mesh=_MESH,
    scratch_types=[
        pltpu.VMEM((ROWS, CH), jnp.int32),
        pltpu.VMEM((ROWS, CH), jnp.int32),
        pltpu.VMEM((ROWS, CH), jnp.int32),
        pltpu.VMEM((CH, H), jnp.float32),
        pltpu.VMEM((CH, H), jnp.float32),
        pltpu.SemaphoreType.DMA,
        pltpu.VMEM_SHARED((ACCR, H), jnp.float32),
    ],
)(_edge_body)


def _scale_body(node_ref, deg_ref, ns0_ref, ns1_ref, nt0_ref, nt1_ref,
                inv_ref):
    deg = deg_ref[...][:, :, :16]
    inv = jnp.where(deg > 0, lax.rsqrt(deg), 0.0)
    node = node_ref[...]
    ns = node * inv[0, :, 0:1]
    nt = node * inv[1, :, 0:1]
    ns0_ref[...] = ns[:, :H]
    ns1_ref[...] = ns[:, H:]
    nt0_ref[...] = nt[:, :H]
    nt1_ref[...] = nt[:, H:]
    inv_ref[...] = inv


def _scale_kernel(node_pad, deg):
    return pl.pallas_call(
        _scale_body,
        grid=(NPAD // NB,),
        in_specs=[
            pl.BlockSpec((NB, D), lambda i: (i, 0)),
            pl.BlockSpec((2, NB, H), lambda i: (0, i, 0)),
        ],
        out_specs=[
            pl.BlockSpec((NB, H), lambda i: (i, 0)),
            pl.BlockSpec((NB, H), lambda i: (i, 0)),
            pl.BlockSpec((NB, H), lambda i: (i, 0)),
            pl.BlockSpec((NB, H), lambda i: (i, 0)),
            pl.BlockSpec((2, NB, 16), lambda i: (0, i, 0)),
        ],
        out_shape=[
            jax.ShapeDtypeStruct((NPAD, H), jnp.float32),
            jax.ShapeDtypeStruct((NPAD, H), jnp.float32),
            jax.ShapeDtypeStruct((NPAD, H), jnp.float32),
            jax.ShapeDtypeStruct((NPAD, H), jnp.float32),
            jax.ShapeDtypeStruct((2, NPAD, 16), jnp.float32),
        ],
    )(node_pad, deg)


def _combine_body(pre_ref, node_ref, inv_ref, wo_ref, wi_ref, wl_ref,
                  lr_ref, b_ref, out_ref):
    p = pre_ref[...]
    inv = inv_ref[...]
    sinv = inv[0, :, 0:1]
    tinv = inv[1, :, 0:1]
    wo = wo_ref[...]
    wi = wi_ref[...]
    f32 = jnp.float32
    ao = (jnp.dot(p[0, 0, 0] + p[0, 1, 0], wo[:H], preferred_element_type=f32)
          + jnp.dot(p[0, 0, 1] + p[0, 1, 1], wo[H:], preferred_element_type=f32))
    ai = (jnp.dot(p[1, 0, 0] + p[1, 1, 0], wi[:H], preferred_element_type=f32)
          + jnp.dot(p[1, 0, 1] + p[1, 1, 1], wi[H:], preferred_element_type=f32))
    ml = jnp.dot(node_ref[...] * lr_ref[...], wl_ref[...],
                 preferred_element_type=f32)
    out_ref[...] = (ao * tinv + ai * sinv + ml) / 3.0 + b_ref[...]


def _combine_kernel(pre, node_emb, inv_n, w_out, w_in, w_loop, loop_rel,
                    bias2):
    return pl.pallas_call(
        _combine_body,
        grid=(NPAD // NB,),
        in_specs=[
            pl.BlockSpec((2, 2, 2, NB, H), lambda i: (0, 0, 0, i, 0)),
            pl.BlockSpec((NB, D), lambda i: (i, 0)),
            pl.BlockSpec((2, NB, 16), lambda i: (0, i, 0)),
            pl.BlockSpec((D, D), lambda i: (0, 0)),
            pl.BlockSpec((D, D), lambda i: (0, 0)),
            pl.BlockSpec((D, D), lambda i: (0, 0)),
            pl.BlockSpec((1, D), lambda i: (0, 0)),
            pl.BlockSpec((1, D), lambda i: (0, 0)),
        ],
        out_specs=pl.BlockSpec((NB, D), lambda i: (i, 0)),
        out_shape=jax.ShapeDtypeStruct((NPAD, D), jnp.float32),
    )(pre, node_emb, inv_n, w_out, w_in, w_loop, loop_rel, bias2)


def _rel_body(rel_ref, w_ref, out_ref, r0_ref, r1_ref):
    rel = rel_ref[...]
    out_ref[...] = jnp.dot(rel, w_ref[...], preferred_element_type=jnp.float32)
    r0_ref[...] = rel[:, :H]
    r1_ref[...] = rel[:, H:]


def _rel_kernel(rel_emb, w_rel):
    r = rel_emb.shape[0]
    rb = NBO
    return pl.pallas_call(
        _rel_body,
        grid=(r // rb,),
        in_specs=[
            pl.BlockSpec((rb, D), lambda i: (i, 0)),
            pl.BlockSpec((D, D), lambda i: (0, 0)),
        ],
        out_specs=[
            pl.BlockSpec((rb, D), lambda i: (i, 0)),
            pl.BlockSpec((rb, H), lambda i: (i, 0)),
            pl.BlockSpec((rb, H), lambda i: (i, 0)),
        ],
        out_shape=[
            jax.ShapeDtypeStruct((r, D), jnp.float32),
            jax.ShapeDtypeStruct((r, H), jnp.float32),
            jax.ShapeDtypeStruct((r, H), jnp.float32),
        ],
    )(rel_emb, w_rel)


def kernel(node_emb, rel_emb, edge_index, W_loop, W_in, W_out, W_rel,
           loop_rel, bias):
    s = edge_index[:, 0]
    r = edge_index[:, 1]
    t = edge_index[:, 2]
    npad_e = EPAD - E
    pad_node = jnp.full((npad_e,), N, jnp.int32)
    pad_rel = jnp.zeros((npad_e,), jnp.int32)
    s2 = jnp.concatenate([s, pad_node]).reshape(EPAD // CH, CH)
    r2 = jnp.concatenate([r, pad_rel]).reshape(EPAD // CH, CH)
    t2 = jnp.concatenate([t, pad_node]).reshape(EPAD // CH, CH)

    node_pad = jnp.zeros((NPAD, D), jnp.float32).at[:N].set(node_emb)
    zeros_h = jnp.zeros((ACCR // 16, H), jnp.float32)

    deg = _deg_kernel(s2, t2, zeros_h)
    ns0, ns1, nt0, nt1, inv = _scale_kernel(node_pad, deg)
    updated_rel, rel0, rel1 = _rel_kernel(rel_emb, W_rel)
    pre = _edge_kernel(ns0, ns1, nt0, nt1, rel0, rel1, s2, r2, t2, zeros_h)
    out = _combine_kernel(pre, node_pad, inv, W_out, W_in, W_loop,
                          loop_rel, bias.reshape(1, D))
    return (out[:N], updated_rel)
